# baseline (speedup 1.0000x reference)
import jax
import jax.numpy as jnp
from jax import lax
from jax.experimental import pallas as pl
from jax.experimental.pallas import tpu as pltpu

H = 376
W = 376
HW = H * W
B = 4
BR = 128
PB = 48128
NB = (H + BR - 1) // BR
CODE = 8
QUAD = 4
LOC_WEIGHT = 2.0
FOCAL_ALPHA = 0.25


def _loss_kernel(t_ref, cls_ref, tf_ref, bp_ref, hbl_ref, sp_ref, ql_ref, out_ref):
    s = pl.program_id(0)

    @pl.when(s == 0)
    def _init():
        for i in range(6):
            out_ref[i] = 0.0

    t = t_ref[...]
    rowok = (lax.broadcasted_iota(jnp.int32, (BR, W), 0) + s * BR) < H
    pos = (t > 0.0) & rowok
    m = pos | ((t == 0.0) & rowok)

    m_cnt = jnp.sum(m.astype(jnp.float32))
    n_pos = jnp.sum(pos.astype(jnp.float32))

    x = cls_ref[...]
    tb = t[None, :, :]
    z = jnp.exp(-jnp.abs(x))
    p = jnp.where(x >= 0.0, 1.0 / (1.0 + z), z / (1.0 + z))
    ce = jnp.maximum(x, 0.0) - x * tb + jnp.log(1.0 + z)
    p_t = p * tb + (1.0 - p) * (1.0 - tb)
    alpha_t = FOCAL_ALPHA * tb + (1.0 - FOCAL_ALPHA) * (1.0 - tb)
    om = 1.0 - p_t
    focal = alpha_t * om * om * ce
    s_focal = jnp.sum(jnp.where(m[None, :, :], focal, 0.0))

    tf = tf_ref[...]
    inb = (lax.iota(jnp.int32, PB) + s * PB) < HW
    mflat = ((tf > 0.0) & inb)[None, :]

    hbl = hbl_ref[...]
    hbls = hbl[0] + hbl[1] + hbl[2] + hbl[3]
    diff = bp_ref[...] - hbls
    ad = jnp.abs(diff)
    sl1 = jnp.where(ad < 1.0, 0.5 * diff * diff, ad - 0.5)
    s_sl1 = jnp.sum(jnp.where(mflat, sl1, 0.0))

    ql = ql_ref[...]
    qls = ql[0] + ql[1] + ql[2] + ql[3]
    spv = sp_ref[...]
    bce = (jnp.maximum(spv, 0.0) - spv * qls
           + jnp.log(1.0 + jnp.exp(-jnp.abs(spv))))
    s_bce = jnp.sum(jnp.where(mflat, bce, 0.0))

    out_ref[0] += s_focal
    out_ref[1] += m_cnt
    out_ref[2] += n_pos
    out_ref[3] += s_sl1
    out_ref[4] += s_bce

    @pl.when(s == NB - 1)
    def _finish():
        cls_loss = out_ref[0] / jnp.maximum(out_ref[1], 1.0)
        reg_loss = out_ref[3] / jnp.maximum(out_ref[2], 1.0) * LOC_WEIGHT
        spa_loss = out_ref[4] / jnp.maximum(out_ref[2] * QUAD, 1.0)
        out_ref[5] = cls_loss + reg_loss + spa_loss


def kernel(cls_preds, box_preds, spa_preds, heatmaps, hos_box_labels, quadrant_labels):
    t2 = heatmaps[0, 0]
    tflat = t2.reshape(HW)
    cls3 = cls_preds.reshape(B, H, W)
    bpT = box_preds.T
    hblT = jnp.transpose(hos_box_labels, (0, 1, 3, 2)).reshape(B, CODE, HW)
    spT = spa_preds.T
    qlT = jnp.transpose(quadrant_labels, (0, 1, 3, 2)).reshape(B, QUAD, HW)

    out = pl.pallas_call(
        _loss_kernel,
        grid=(NB,),
        in_specs=[
            pl.BlockSpec((BR, W), lambda s: (s, 0)),
            pl.BlockSpec((B, BR, W), lambda s: (0, s, 0)),
            pl.BlockSpec((PB,), lambda s: (s,)),
            pl.BlockSpec((CODE, PB), lambda s: (0, s)),
            pl.BlockSpec((B, CODE, PB), lambda s: (0, 0, s)),
            pl.BlockSpec((QUAD, PB), lambda s: (0, s)),
            pl.BlockSpec((B, QUAD, PB), lambda s: (0, 0, s)),
        ],
        out_specs=pl.BlockSpec(memory_space=pltpu.SMEM),
        out_shape=jax.ShapeDtypeStruct((6,), jnp.float32),
    )(t2, cls3, tflat, bpT, hblT, spT, qlT)
    return out[5]


# speedup vs baseline: 1.0180x; 1.0180x over previous
import jax
import jax.numpy as jnp
from jax import lax
from jax.experimental import pallas as pl
from jax.experimental.pallas import tpu as pltpu

H = 376
W = 376
HW = H * W
B = 4
BR = 96
PB = 35840
NB = (H + BR - 1) // BR
CODE = 8
QUAD = 4
LOC_WEIGHT = 2.0
FOCAL_ALPHA = 0.25


def _loss_kernel(t_ref, cls_ref, tf_ref, bp_ref, hbl_ref, sp_ref, ql_ref, out_ref):
    s = pl.program_id(0)

    @pl.when(s == 0)
    def _init():
        for i in range(6):
            out_ref[i] = 0.0

    t = t_ref[...]
    rowok = (lax.broadcasted_iota(jnp.int32, (BR, W), 0) + s * BR) < H
    pos = (t > 0.0) & rowok
    m = pos | ((t == 0.0) & rowok)

    m_cnt = jnp.sum(m.astype(jnp.float32))
    n_pos = jnp.sum(pos.astype(jnp.float32))

    x = cls_ref[...]
    tb = t[None, :, :]
    z = jnp.exp(-jnp.abs(x))
    p = jnp.where(x >= 0.0, 1.0 / (1.0 + z), z / (1.0 + z))
    ce = jnp.maximum(x, 0.0) - x * tb + jnp.log(1.0 + z)
    p_t = p * tb + (1.0 - p) * (1.0 - tb)
    alpha_t = FOCAL_ALPHA * tb + (1.0 - FOCAL_ALPHA) * (1.0 - tb)
    om = 1.0 - p_t
    focal = alpha_t * om * om * ce
    s_focal = jnp.sum(jnp.where(m[None, :, :], focal, 0.0))

    tf = tf_ref[...]
    inb = (lax.iota(jnp.int32, PB) + s * PB) < HW
    mflat = ((tf > 0.0) & inb)[None, :]

    hbl = hbl_ref[...]
    hbls = hbl[0] + hbl[1] + hbl[2] + hbl[3]
    diff = bp_ref[...] - hbls
    ad = jnp.abs(diff)
    sl1 = jnp.where(ad < 1.0, 0.5 * diff * diff, ad - 0.5)
    s_sl1 = jnp.sum(jnp.where(mflat, sl1, 0.0))

    ql = ql_ref[...]
    qls = ql[0] + ql[1] + ql[2] + ql[3]
    spv = sp_ref[...]
    bce = (jnp.maximum(spv, 0.0) - spv * qls
           + jnp.log(1.0 + jnp.exp(-jnp.abs(spv))))
    s_bce = jnp.sum(jnp.where(mflat, bce, 0.0))

    out_ref[0] += s_focal
    out_ref[1] += m_cnt
    out_ref[2] += n_pos
    out_ref[3] += s_sl1
    out_ref[4] += s_bce

    @pl.when(s == NB - 1)
    def _finish():
        cls_loss = out_ref[0] / jnp.maximum(out_ref[1], 1.0)
        reg_loss = out_ref[3] / jnp.maximum(out_ref[2], 1.0) * LOC_WEIGHT
        spa_loss = out_ref[4] / jnp.maximum(out_ref[2] * QUAD, 1.0)
        out_ref[5] = cls_loss + reg_loss + spa_loss


def kernel(cls_preds, box_preds, spa_preds, heatmaps, hos_box_labels, quadrant_labels):
    t2 = heatmaps[0, 0]
    tflat = t2.reshape(HW)
    cls3 = cls_preds.reshape(B, H, W)
    bpT = box_preds.T
    hblT = jnp.transpose(hos_box_labels, (0, 1, 3, 2)).reshape(B, CODE, HW)
    spT = spa_preds.T
    qlT = jnp.transpose(quadrant_labels, (0, 1, 3, 2)).reshape(B, QUAD, HW)

    out = pl.pallas_call(
        _loss_kernel,
        grid=(NB,),
        in_specs=[
            pl.BlockSpec((BR, W), lambda s: (s, 0)),
            pl.BlockSpec((B, BR, W), lambda s: (0, s, 0)),
            pl.BlockSpec((PB,), lambda s: (s,)),
            pl.BlockSpec((CODE, PB), lambda s: (0, s)),
            pl.BlockSpec((B, CODE, PB), lambda s: (0, 0, s)),
            pl.BlockSpec((QUAD, PB), lambda s: (0, s)),
            pl.BlockSpec((B, QUAD, PB), lambda s: (0, 0, s)),
        ],
        out_specs=pl.BlockSpec(memory_space=pltpu.SMEM),
        out_shape=jax.ShapeDtypeStruct((6,), jnp.float32),
    )(t2, cls3, tflat, bpT, hblT, spT, qlT)
    return out[5]
